# baseline (speedup 1.0000x reference)
import numpy as np
import jax
import jax.numpy as jnp
from jax import lax
from jax.experimental import pallas as pl
from jax.experimental.pallas import tpu as pltpu
from jax.experimental.pallas import tpu_sc as plsc

_CBV = [0.25, 0.3536, 0.5, 0.7071, 1.0, 1.4142, 2.0, 2.8284, 4.0]

_EPS = np.array([float.fromhex(s) for s in [
    '0x1.1d32320000000p+0', '0x1.220f960000000p-3', '-0x1.0b7f1e0000000p-1', '-0x1.bb707a0000000p-2',
    '0x1.34ccc20000000p+1', '-0x1.a498300000000p-1', '-0x1.6980680000000p-3', '-0x1.d443fc0000000p-1',
    '0x1.04b9c00000000p+0', '-0x1.e2555e0000000p+0', '-0x1.2f99500000000p+0', '0x1.fa88fa0000000p-1',
    '0x1.0dff540000000p-1', '-0x1.0f317c0000000p+0', '0x1.00a7520000000p-1', '0x1.af737c0000000p-2',
    '0x1.c92e1a0000000p-2', '-0x1.ce80ce0000000p-1', '-0x1.992bbc0000000p-5', '0x1.bcfd6e0000000p-2',
    '0x1.9a4e140000000p-4', '0x1.1a05440000000p+0', '-0x1.99b1b00000000p-2', '0x1.23e0160000000p+0',
    '0x1.fca9bc0000000p-2', '-0x1.066bfc0000000p+0', '-0x1.3d9a420000000p-1', '0x1.efa56a0000000p+0',
    '0x1.d108600000000p-2', '0x1.e0d9f20000000p-1', '-0x1.1365d40000000p-2', '-0x1.86de460000000p-3',
    '-0x1.9daada0000000p-1', '0x1.dd73f60000000p-5', '0x1.adc7940000000p-1', '0x1.97004e0000000p-1',
    '0x1.0f76ae0000000p+1', '-0x1.8a21760000000p+0', '-0x1.b970be0000000p+0', '0x1.74feca0000000p-5',
    '-0x1.2c73580000000p+0', '0x1.9f54820000000p-1', '0x1.1f2e720000000p-2', '0x1.172baa0000000p+0',
    '0x1.76097c0000000p-2', '-0x1.86285e0000000p+0', '0x1.0cb2080000000p-1', '0x1.fa0dfc0000000p-2',
    '0x1.b1f70e0000000p-1', '0x1.e7daf20000000p+0', '0x1.f1bafc0000000p-5', '0x1.d95e9c0000000p-3',
    '0x1.d9dba60000000p-3', '-0x1.4f5dec0000000p-1', '-0x1.8395f40000000p-4', '0x1.59864a0000000p-1',
    '-0x1.a409a80000000p-1', '-0x1.4bdb600000000p-1', '-0x1.cca4740000000p+0', '0x1.04ee680000000p+1',
    '-0x1.ce4f740000000p-1', '-0x1.1cd6aa0000000p-6', '-0x1.455f560000000p-1', '0x1.24f20a0000000p+0',
    '-0x1.5fc1e00000000p+0', '0x1.3f35880000000p-1', '-0x1.da5c520000000p+0', '0x1.70f80a0000000p-1',
    '0x1.25eaa60000000p-2', '0x1.093ebe0000000p-1', '-0x1.01849a0000000p+0', '0x1.024cfa0000000p+0',
    '0x1.06bd420000000p-2', '0x1.9b49ea0000000p-3', '-0x1.f441ee0000000p+0', '0x1.2927740000000p-2',
    '-0x1.7fc5840000000p-2', '-0x1.ed83d00000000p-1', '0x1.18f3080000000p-1', '-0x1.4c742c0000000p-2',
    '0x1.eddea80000000p-2', '-0x1.2066040000000p+0', '-0x1.e3ffa80000000p+0', '-0x1.8c53f40000000p+0',
    '-0x1.780b300000000p-1', '-0x1.e8dafc0000000p-2', '0x1.9fccf20000000p-1', '0x1.64cfe40000000p-4',
    '0x1.8c97e20000000p+0', '0x1.28459a0000000p-1', '-0x1.63208c0000000p-3', '-0x1.1212a20000000p+0',
    '-0x1.fccda40000000p-3', '-0x1.fc41be0000000p-2', '0x1.fdecdc0000000p-3', '-0x1.b11a1e0000000p-1',
    '0x1.a9ffdc0000000p-1', '-0x1.6107760000000p-5', '0x1.20bfb60000000p-1', '-0x1.5beb420000000p+0',
    '-0x1.47ec420000000p-1', '-0x1.2186420000000p+0', '0x1.6277f00000000p-1', '0x1.3e564a0000000p+0',
    '0x1.3114260000000p-1', '0x1.5d0c600000000p-3', '-0x1.24dda40000000p+0', '-0x1.eb355a0000000p+0',
    '0x1.6701540000000p+0', '0x1.0dd0c20000000p+0', '-0x1.4208e40000000p+0', '0x1.1341fe0000000p+0',
]], dtype=np.float32).reshape(4, 28)

_EPS_O = 0
_B1_O = 128
_W1_O = 240
_B2M_O = 5616
_W2M_O = 5648
_B2S_O = 9232
_W2S_O = 9264
_B3_O = 12848
_W3_O = 12960
_B4_O = 18336
_W4_O = 18368
_B5_O = 21952
_W5_O = 22064
_B6M_O = 25648
_W6M_O = 25680
_MEM_N = 29264
_BLOB_N = 48 + _MEM_N


def _body(blob_h, out_h, mem, vin, h, outv, sem):
    cid = lax.axis_index("c")
    sid = lax.axis_index("s")

    @pl.when(jnp.logical_and(cid == 0, sid == 0))
    def _():
        iota = lax.iota(jnp.int32, 16)
        tail12 = iota < 12

        ca = pltpu.async_copy(blob_h.at[pl.ds(0, 48)], vin.at[pl.ds(0, 48)],
                              sem)
        cb = pltpu.async_copy(blob_h.at[pl.ds(48, _MEM_N)], mem, sem)
        ca.wait()
        cb.wait()

        def mv(woff, boff, nb, ld, src, cbase, nchunks):
            accs = tuple(mem[pl.ds(boff + o * 16, 16)] for o in range(nb))

            def cbody(c, accs):
                chunk = src[pl.ds(cbase + c * 16, 16)]
                out = accs
                for jl in range(16):
                    bv = jnp.take(chunk, jnp.full((16,), jl, jnp.int32))
                    row = woff + (c * 16 + jl) * ld
                    out = tuple(out[o] + bv * mem[pl.ds(row + o * 16, 16)]
                                for o in range(nb))
                return out

            return lax.fori_loop(0, nchunks, cbody, accs)

        def sigm(v):
            return 1.0 / (1.0 + jnp.exp(-v))

        def quant(zv):
            bd = jnp.abs(zv - _CBV[0])
            bv = jnp.full((16,), _CBV[0], dtype=jnp.float32)
            for c in _CBV[1:]:
                d = jnp.abs(zv - jnp.float32(c))
                t = d < bd
                bd = jnp.where(t, d, bd)
                bv = jnp.where(t, jnp.float32(c), bv)
            return bv

        def encode_body(i, carry):
            hb = mv(_W1_O, _B1_O, 7, 112, vin, 0, 3)
            for o in range(7):
                h[pl.ds(o * 16, 16)] = jnp.maximum(hb[o], 0.0)

            def w2p(pi, c):
                woff = jnp.where(pi == 0, _W2M_O, _W2S_O)
                boff = jnp.where(pi == 0, _B2M_O, _B2S_O)
                a0, a1 = mv(woff, boff, 2, 32, h, 0, 7)
                p = jnp.full((16,), pi, jnp.int32) == 0
                m0, m1, s0, s1 = c
                return (jnp.where(p, a0, m0), jnp.where(p, a1, m1),
                        jnp.where(p, s0, a0), jnp.where(p, s1, a1))

            zv = jnp.zeros((16,), jnp.float32)
            mu0, mu1, s0, s1 = lax.fori_loop(0, 2, w2p, (zv, zv, zv, zv))
            ls0, ls1 = sigm(s0), sigm(s1)
            mi = jnp.full((16,), i, jnp.int32) < 3
            mbase = i * 28
            plsc.store_scatter(outv, [mbase + iota], mu0, mask=mi)
            plsc.store_scatter(outv, [mbase + 16 + iota], mu1,
                               mask=jnp.logical_and(mi, tail12))
            lbase = 112 + i * 28
            plsc.store_scatter(outv, [lbase + iota], ls0, mask=mi)
            plsc.store_scatter(outv, [lbase + 16 + iota], ls1,
                               mask=jnp.logical_and(mi, tail12))
            e0 = mem[pl.ds(_EPS_O + i * 32, 16)]
            e1 = mem[pl.ds(_EPS_O + i * 32 + 16, 16)]
            z0, z1 = mu0 + e0 * ls0, mu1 + e1 * ls1
            vin[pl.ds(0, 16)] = quant(z0)
            plsc.store_scatter(vin, [16 + iota], quant(z1), mask=tail12)
            return carry

        lax.fori_loop(0, 4, encode_body, 0)

        zero = jnp.zeros((16,), dtype=jnp.float32)
        outv[pl.ds(84, 16)] = zero
        plsc.store_scatter(outv, [100 + iota], zero, mask=tail12)
        outv[pl.ds(112 + 84, 16)] = zero
        plsc.store_scatter(outv, [112 + 100 + iota], zero, mask=tail12)

        cma = pltpu.async_copy(outv.at[pl.ds(0, 224)], out_h.at[pl.ds(0, 224)],
                               sem)

        def dec_body(t, carry):
            k = 3 - t

            def half(hi, c):
                woff7 = jnp.where(hi == 0, _W3_O, _W5_O)
                boff7 = jnp.where(hi == 0, _B3_O, _B5_O)
                cbase = jnp.where(hi == 0, 0, 48)
                nch = jnp.where(hi == 0, 3, 2)
                hb = mv(woff7, boff7, 7, 112, vin, cbase, nch)
                relu0 = jnp.full((16,), hi, jnp.int32) == 0
                for o in range(7):
                    h[pl.ds(o * 16, 16)] = jnp.where(
                        relu0, jnp.maximum(hb[o], 0.0), hb[o])
                woff2 = jnp.where(hi == 0, _W4_O, _W6M_O)
                boff2 = jnp.where(hi == 0, _B4_O, _B6M_O)
                a0, a1 = mv(woff2, boff2, 2, 32, h, 0, 7)

                @pl.when(hi == 0)
                def _():
                    r0, r1 = sigm(a0), sigm(a1)
                    vin[pl.ds(48, 16)] = r0
                    vin[pl.ds(64, 16)] = r1
                    outv[pl.ds(336, 16)] = r0
                    plsc.store_scatter(outv, [352 + iota], r1, mask=tail12)
                    vin[pl.ds(0, 16)] = quant(r0)
                    plsc.store_scatter(vin, [16 + iota], quant(r1),
                                       mask=tail12)

                @pl.when(hi == 1)
                def _():
                    base = 224 + k * 28
                    plsc.store_scatter(outv, [base + iota], a0)
                    plsc.store_scatter(outv, [base + 16 + iota], a1,
                                       mask=tail12)

                return c

            lax.fori_loop(0, 2, half, 0)
            return carry

        lax.fori_loop(0, 4, dec_body, 0)

        cmb = pltpu.async_copy(outv.at[pl.ds(224, 144)],
                               out_h.at[pl.ds(224, 144)], sem)
        cma.wait()
        cmb.wait()


_MESH = plsc.VectorSubcoreMesh(core_axis_name="c", subcore_axis_name="s",
                               num_cores=1, num_subcores=1)

_call = pl.kernel(
    _body,
    out_type=[jax.ShapeDtypeStruct((368,), jnp.float32)],
    mesh=_MESH,
    compiler_params=pltpu.CompilerParams(use_tc_tiling_on_sc=False,
                                         needs_layout_passes=False,
                                         disable_bounds_checks=True),
    scratch_types=[
        pltpu.VMEM((_MEM_N,), jnp.float32),
        pltpu.VMEM((80,), jnp.float32),
        pltpu.VMEM((112,), jnp.float32),
        pltpu.VMEM((368,), jnp.float32),
        pltpu.SemaphoreType.DMA,
    ],
)


def _tp(W, nrows, ncols):
    out_d, in_d = W.shape
    return jnp.pad(W.T, ((0, nrows - in_d), (0, ncols - out_d))).reshape(-1)


def _tp_cat(W):
    Wt = W.T
    z4 = jnp.zeros((4, 100), jnp.float32)
    z6 = jnp.zeros((6, 100), jnp.float32)
    Wv = jnp.concatenate([Wt[:28], z4, Wt[28:], z6], axis=0)
    return jnp.pad(Wv, ((0, 0), (0, 12))).reshape(-1)


def _padv(v, n):
    return jnp.pad(v, (0, n - v.shape[0]))


def kernel(x, y, params):
    p = params
    blob = jnp.concatenate([
        x, jnp.zeros((4,), jnp.float32), y, jnp.zeros((6,), jnp.float32),
        jnp.asarray(np.pad(_EPS, ((0, 0), (0, 4))).reshape(-1)),
        _padv(p['b1'], 112), _tp_cat(p['W1']),
        _padv(p['b2m'], 32), _tp(p['W2m'], 112, 32),
        _padv(p['b2s'], 32), _tp(p['W2s'], 112, 32),
        _padv(p['b3'], 112), _tp_cat(p['W3']),
        _padv(p['b4'], 32), _tp(p['W4'], 112, 32),
        _padv(p['b5'], 112), _tp(p['W5'], 32, 112),
        _padv(p['b6m'], 32), _tp(p['W6m'], 112, 32),
    ])
    (o,) = _call(blob)
    return (o[336:364], o[0:112].reshape(4, 28), o[224:336].reshape(4, 28),
            o[112:224].reshape(4, 28))


# speedup vs baseline: 1.0136x; 1.0136x over previous
import numpy as np
import jax
import jax.numpy as jnp
from jax import lax
from jax.experimental import pallas as pl
from jax.experimental.pallas import tpu as pltpu
from jax.experimental.pallas import tpu_sc as plsc

_CBV = [0.25, 0.3536, 0.5, 0.7071, 1.0, 1.4142, 2.0, 2.8284, 4.0]

_EPS = np.array([float.fromhex(s) for s in [
    '0x1.1d32320000000p+0', '0x1.220f960000000p-3', '-0x1.0b7f1e0000000p-1', '-0x1.bb707a0000000p-2',
    '0x1.34ccc20000000p+1', '-0x1.a498300000000p-1', '-0x1.6980680000000p-3', '-0x1.d443fc0000000p-1',
    '0x1.04b9c00000000p+0', '-0x1.e2555e0000000p+0', '-0x1.2f99500000000p+0', '0x1.fa88fa0000000p-1',
    '0x1.0dff540000000p-1', '-0x1.0f317c0000000p+0', '0x1.00a7520000000p-1', '0x1.af737c0000000p-2',
    '0x1.c92e1a0000000p-2', '-0x1.ce80ce0000000p-1', '-0x1.992bbc0000000p-5', '0x1.bcfd6e0000000p-2',
    '0x1.9a4e140000000p-4', '0x1.1a05440000000p+0', '-0x1.99b1b00000000p-2', '0x1.23e0160000000p+0',
    '0x1.fca9bc0000000p-2', '-0x1.066bfc0000000p+0', '-0x1.3d9a420000000p-1', '0x1.efa56a0000000p+0',
    '0x1.d108600000000p-2', '0x1.e0d9f20000000p-1', '-0x1.1365d40000000p-2', '-0x1.86de460000000p-3',
    '-0x1.9daada0000000p-1', '0x1.dd73f60000000p-5', '0x1.adc7940000000p-1', '0x1.97004e0000000p-1',
    '0x1.0f76ae0000000p+1', '-0x1.8a21760000000p+0', '-0x1.b970be0000000p+0', '0x1.74feca0000000p-5',
    '-0x1.2c73580000000p+0', '0x1.9f54820000000p-1', '0x1.1f2e720000000p-2', '0x1.172baa0000000p+0',
    '0x1.76097c0000000p-2', '-0x1.86285e0000000p+0', '0x1.0cb2080000000p-1', '0x1.fa0dfc0000000p-2',
    '0x1.b1f70e0000000p-1', '0x1.e7daf20000000p+0', '0x1.f1bafc0000000p-5', '0x1.d95e9c0000000p-3',
    '0x1.d9dba60000000p-3', '-0x1.4f5dec0000000p-1', '-0x1.8395f40000000p-4', '0x1.59864a0000000p-1',
    '-0x1.a409a80000000p-1', '-0x1.4bdb600000000p-1', '-0x1.cca4740000000p+0', '0x1.04ee680000000p+1',
    '-0x1.ce4f740000000p-1', '-0x1.1cd6aa0000000p-6', '-0x1.455f560000000p-1', '0x1.24f20a0000000p+0',
    '-0x1.5fc1e00000000p+0', '0x1.3f35880000000p-1', '-0x1.da5c520000000p+0', '0x1.70f80a0000000p-1',
    '0x1.25eaa60000000p-2', '0x1.093ebe0000000p-1', '-0x1.01849a0000000p+0', '0x1.024cfa0000000p+0',
    '0x1.06bd420000000p-2', '0x1.9b49ea0000000p-3', '-0x1.f441ee0000000p+0', '0x1.2927740000000p-2',
    '-0x1.7fc5840000000p-2', '-0x1.ed83d00000000p-1', '0x1.18f3080000000p-1', '-0x1.4c742c0000000p-2',
    '0x1.eddea80000000p-2', '-0x1.2066040000000p+0', '-0x1.e3ffa80000000p+0', '-0x1.8c53f40000000p+0',
    '-0x1.780b300000000p-1', '-0x1.e8dafc0000000p-2', '0x1.9fccf20000000p-1', '0x1.64cfe40000000p-4',
    '0x1.8c97e20000000p+0', '0x1.28459a0000000p-1', '-0x1.63208c0000000p-3', '-0x1.1212a20000000p+0',
    '-0x1.fccda40000000p-3', '-0x1.fc41be0000000p-2', '0x1.fdecdc0000000p-3', '-0x1.b11a1e0000000p-1',
    '0x1.a9ffdc0000000p-1', '-0x1.6107760000000p-5', '0x1.20bfb60000000p-1', '-0x1.5beb420000000p+0',
    '-0x1.47ec420000000p-1', '-0x1.2186420000000p+0', '0x1.6277f00000000p-1', '0x1.3e564a0000000p+0',
    '0x1.3114260000000p-1', '0x1.5d0c600000000p-3', '-0x1.24dda40000000p+0', '-0x1.eb355a0000000p+0',
    '0x1.6701540000000p+0', '0x1.0dd0c20000000p+0', '-0x1.4208e40000000p+0', '0x1.1341fe0000000p+0',
]], dtype=np.float32).reshape(4, 28)

_EPS_O = 0
_B1_O = 128
_W1_O = 240
_B2M_O = 5616
_W2M_O = 5648
_B2S_O = 8848
_W2S_O = 8880
_B3_O = 12080
_W3_O = 12192
_B4_O = 17568
_W4_O = 17600
_B5_O = 20800
_W5_O = 20912
_B6M_O = 24048
_W6M_O = 24080
_MEM_N = 27280
_BLOB_N = 48 + _MEM_N


def _body(blob_h, out_h, mem, vin, h, r, outv, sem):
    cid = lax.axis_index("c")
    sid = lax.axis_index("s")

    @pl.when(jnp.logical_and(cid == 0, sid == 0))
    def _():
        iota = lax.iota(jnp.int32, 16)
        tail12 = iota < 12

        ca = pltpu.async_copy(blob_h.at[pl.ds(0, 48)], vin, sem)
        cb = pltpu.async_copy(blob_h.at[pl.ds(48, _MEM_N)], mem, sem)
        ca.wait()
        cb.wait()

        def mm(weights, src, src_n):
            accs = []
            for _w, boff, out_d, _ld in weights:
                nb = (out_d + 15) // 16
                accs += [mem[pl.ds(boff + o * 16, 16)] for o in range(nb)]

            def step(accs, chunk, jl, j):
                bv = jnp.take(chunk, jnp.full((16,), jl, jnp.int32))
                out, k = [], 0
                for woff, _b, out_d, ld in weights:
                    nb = (out_d + 15) // 16
                    row = woff + j * ld
                    for o in range(nb):
                        out.append(accs[k] + bv * mem[pl.ds(row + o * 16, 16)])
                        k += 1
                return tuple(out)

            nchunks, tail = divmod(src_n, 16)

            def cbody(c, accs):
                base = c * 16
                chunk = src[pl.ds(base, 16)]
                for jl in range(16):
                    accs = step(accs, chunk, jl, base + jl)
                return accs

            accs = lax.fori_loop(0, nchunks, cbody, tuple(accs))
            if tail:
                base = nchunks * 16
                chunk = src[pl.ds(base, 16)]
                for jl in range(tail):
                    accs = step(accs, chunk, jl, base + jl)
            res, k = [], 0
            for _w, _b, out_d, _ld in weights:
                nb = (out_d + 15) // 16
                res.append(accs[k:k + nb])
                k += nb
            return res

        def sigm(v):
            return 1.0 / (1.0 + jnp.exp(-v))

        def quant(zv):
            bd = jnp.abs(zv - _CBV[0])
            bv = jnp.full((16,), _CBV[0], dtype=jnp.float32)
            for c in _CBV[1:]:
                d = jnp.abs(zv - jnp.float32(c))
                t = d < bd
                bd = jnp.where(t, d, bd)
                bv = jnp.where(t, jnp.float32(c), bv)
            return bv

        def encode_body(i, carry):
            (hb,) = mm([(_W1_O, _B1_O, 100, 112)], vin, 42)
            for o in range(7):
                h[pl.ds(o * 16, 16)] = jnp.maximum(hb[o], 0.0)
            mres = mm([(_W2M_O, _B2M_O, 28, 32), (_W2S_O, _B2S_O, 28, 32)],
                      h, 100)
            mu0, mu1 = mres[0]
            ls0, ls1 = sigm(mres[1][0]), sigm(mres[1][1])
            mi = jnp.full((16,), i, jnp.int32) < 3
            mbase = 28 + i * 28
            plsc.store_scatter(outv, [mbase + iota], mu0, mask=mi)
            plsc.store_scatter(outv, [mbase + 16 + iota], mu1,
                               mask=jnp.logical_and(mi, tail12))
            lbase = 252 + i * 28
            plsc.store_scatter(outv, [lbase + iota], ls0, mask=mi)
            plsc.store_scatter(outv, [lbase + 16 + iota], ls1,
                               mask=jnp.logical_and(mi, tail12))
            e0 = mem[pl.ds(_EPS_O + i * 32, 16)]
            e1 = mem[pl.ds(_EPS_O + i * 32 + 16, 16)]
            z0, z1 = mu0 + e0 * ls0, mu1 + e1 * ls1
            vin[pl.ds(0, 16)] = quant(z0)
            plsc.store_scatter(vin, [16 + iota], quant(z1), mask=tail12)
            return carry

        lax.fori_loop(0, 4, encode_body, 0)

        zero = jnp.zeros((16,), dtype=jnp.float32)
        outv[pl.ds(28 + 84, 16)] = zero
        plsc.store_scatter(outv, [28 + 100 + iota], zero, mask=tail12)
        outv[pl.ds(252 + 84, 16)] = zero
        plsc.store_scatter(outv, [252 + 100 + iota], zero, mask=tail12)

        def dec_body(t, carry):
            k = 3 - t
            (hb,) = mm([(_W3_O, _B3_O, 100, 112)], vin, 42)
            for o in range(7):
                h[pl.ds(o * 16, 16)] = jnp.maximum(hb[o], 0.0)
            ((r0, r1),) = mm([(_W4_O, _B4_O, 28, 32)], h, 100)
            r0, r1 = sigm(r0), sigm(r1)
            r[pl.ds(0, 16)] = r0
            r[pl.ds(16, 16)] = r1
            outv[pl.ds(0, 16)] = r0
            plsc.store_scatter(outv, [16 + iota], r1, mask=tail12)
            vin[pl.ds(0, 16)] = quant(r0)
            plsc.store_scatter(vin, [16 + iota], quant(r1), mask=tail12)
            (hb5,) = mm([(_W5_O, _B5_O, 100, 112)], r, 28)
            for o in range(7):
                h[pl.ds(o * 16, 16)] = hb5[o]
            ((m0, m1),) = mm([(_W6M_O, _B6M_O, 28, 32)], h, 100)
            base = 140 + k * 28
            plsc.store_scatter(outv, [base + iota], m0)
            plsc.store_scatter(outv, [base + 16 + iota], m1, mask=tail12)
            return carry

        lax.fori_loop(0, 4, dec_body, 0)

        pltpu.async_copy(outv, out_h, sem).wait()


_MESH = plsc.VectorSubcoreMesh(core_axis_name="c", subcore_axis_name="s",
                               num_cores=1, num_subcores=1)

_call = pl.kernel(
    _body,
    out_type=[jax.ShapeDtypeStruct((368,), jnp.float32)],
    mesh=_MESH,
    compiler_params=pltpu.CompilerParams(use_tc_tiling_on_sc=False,
                                         needs_layout_passes=False,
                                         disable_bounds_checks=True),
    scratch_types=[
        pltpu.VMEM((_MEM_N,), jnp.float32),
        pltpu.VMEM((48,), jnp.float32),
        pltpu.VMEM((112,), jnp.float32),
        pltpu.VMEM((32,), jnp.float32),
        pltpu.VMEM((368,), jnp.float32),
        pltpu.SemaphoreType.DMA,
    ],
)


def _tp(W, nrows, ncols):
    out_d, in_d = W.shape
    return jnp.pad(W.T, ((0, nrows - in_d), (0, ncols - out_d))).reshape(-1)


def _tp_cat(W):
    Wt = W.T
    z4 = jnp.zeros((4, 100), jnp.float32)
    z6 = jnp.zeros((6, 100), jnp.float32)
    Wv = jnp.concatenate([Wt[:28], z4, Wt[28:], z6], axis=0)
    return jnp.pad(Wv, ((0, 0), (0, 12))).reshape(-1)


def _padv(v, n):
    return jnp.pad(v, (0, n - v.shape[0]))


def kernel(x, y, params):
    p = params
    blob = jnp.concatenate([
        x, jnp.zeros((4,), jnp.float32), y, jnp.zeros((6,), jnp.float32),
        jnp.asarray(np.pad(_EPS, ((0, 0), (0, 4))).reshape(-1)),
        _padv(p['b1'], 112), _tp_cat(p['W1']),
        _padv(p['b2m'], 32), _tp(p['W2m'], 100, 32),
        _padv(p['b2s'], 32), _tp(p['W2s'], 100, 32),
        _padv(p['b3'], 112), _tp_cat(p['W3']),
        _padv(p['b4'], 32), _tp(p['W4'], 100, 32),
        _padv(p['b5'], 112), _tp(p['W5'], 28, 112),
        _padv(p['b6m'], 32), _tp(p['W6m'], 100, 32),
    ])
    (o,) = _call(blob)
    return (o[0:28], o[28:140].reshape(4, 28), o[140:252].reshape(4, 28),
            o[252:364].reshape(4, 28))


# speedup vs baseline: 1.0237x; 1.0100x over previous
import numpy as np
import jax
import jax.numpy as jnp
from jax import lax
from jax.experimental import pallas as pl
from jax.experimental.pallas import tpu as pltpu
from jax.experimental.pallas import tpu_sc as plsc

_CBV = [0.25, 0.3536, 0.5, 0.7071, 1.0, 1.4142, 2.0, 2.8284, 4.0]

_EPS = np.array([float.fromhex(s) for s in [
    '0x1.1d32320000000p+0', '0x1.220f960000000p-3', '-0x1.0b7f1e0000000p-1', '-0x1.bb707a0000000p-2',
    '0x1.34ccc20000000p+1', '-0x1.a498300000000p-1', '-0x1.6980680000000p-3', '-0x1.d443fc0000000p-1',
    '0x1.04b9c00000000p+0', '-0x1.e2555e0000000p+0', '-0x1.2f99500000000p+0', '0x1.fa88fa0000000p-1',
    '0x1.0dff540000000p-1', '-0x1.0f317c0000000p+0', '0x1.00a7520000000p-1', '0x1.af737c0000000p-2',
    '0x1.c92e1a0000000p-2', '-0x1.ce80ce0000000p-1', '-0x1.992bbc0000000p-5', '0x1.bcfd6e0000000p-2',
    '0x1.9a4e140000000p-4', '0x1.1a05440000000p+0', '-0x1.99b1b00000000p-2', '0x1.23e0160000000p+0',
    '0x1.fca9bc0000000p-2', '-0x1.066bfc0000000p+0', '-0x1.3d9a420000000p-1', '0x1.efa56a0000000p+0',
    '0x1.d108600000000p-2', '0x1.e0d9f20000000p-1', '-0x1.1365d40000000p-2', '-0x1.86de460000000p-3',
    '-0x1.9daada0000000p-1', '0x1.dd73f60000000p-5', '0x1.adc7940000000p-1', '0x1.97004e0000000p-1',
    '0x1.0f76ae0000000p+1', '-0x1.8a21760000000p+0', '-0x1.b970be0000000p+0', '0x1.74feca0000000p-5',
    '-0x1.2c73580000000p+0', '0x1.9f54820000000p-1', '0x1.1f2e720000000p-2', '0x1.172baa0000000p+0',
    '0x1.76097c0000000p-2', '-0x1.86285e0000000p+0', '0x1.0cb2080000000p-1', '0x1.fa0dfc0000000p-2',
    '0x1.b1f70e0000000p-1', '0x1.e7daf20000000p+0', '0x1.f1bafc0000000p-5', '0x1.d95e9c0000000p-3',
    '0x1.d9dba60000000p-3', '-0x1.4f5dec0000000p-1', '-0x1.8395f40000000p-4', '0x1.59864a0000000p-1',
    '-0x1.a409a80000000p-1', '-0x1.4bdb600000000p-1', '-0x1.cca4740000000p+0', '0x1.04ee680000000p+1',
    '-0x1.ce4f740000000p-1', '-0x1.1cd6aa0000000p-6', '-0x1.455f560000000p-1', '0x1.24f20a0000000p+0',
    '-0x1.5fc1e00000000p+0', '0x1.3f35880000000p-1', '-0x1.da5c520000000p+0', '0x1.70f80a0000000p-1',
    '0x1.25eaa60000000p-2', '0x1.093ebe0000000p-1', '-0x1.01849a0000000p+0', '0x1.024cfa0000000p+0',
    '0x1.06bd420000000p-2', '0x1.9b49ea0000000p-3', '-0x1.f441ee0000000p+0', '0x1.2927740000000p-2',
    '-0x1.7fc5840000000p-2', '-0x1.ed83d00000000p-1', '0x1.18f3080000000p-1', '-0x1.4c742c0000000p-2',
    '0x1.eddea80000000p-2', '-0x1.2066040000000p+0', '-0x1.e3ffa80000000p+0', '-0x1.8c53f40000000p+0',
    '-0x1.780b300000000p-1', '-0x1.e8dafc0000000p-2', '0x1.9fccf20000000p-1', '0x1.64cfe40000000p-4',
    '0x1.8c97e20000000p+0', '0x1.28459a0000000p-1', '-0x1.63208c0000000p-3', '-0x1.1212a20000000p+0',
    '-0x1.fccda40000000p-3', '-0x1.fc41be0000000p-2', '0x1.fdecdc0000000p-3', '-0x1.b11a1e0000000p-1',
    '0x1.a9ffdc0000000p-1', '-0x1.6107760000000p-5', '0x1.20bfb60000000p-1', '-0x1.5beb420000000p+0',
    '-0x1.47ec420000000p-1', '-0x1.2186420000000p+0', '0x1.6277f00000000p-1', '0x1.3e564a0000000p+0',
    '0x1.3114260000000p-1', '0x1.5d0c600000000p-3', '-0x1.24dda40000000p+0', '-0x1.eb355a0000000p+0',
    '0x1.6701540000000p+0', '0x1.0dd0c20000000p+0', '-0x1.4208e40000000p+0', '0x1.1341fe0000000p+0',
]], dtype=np.float32).reshape(4, 28)

_EPS_O = 0
_B1_O = 128
_W1_O = 240
_B2M_O = 5616
_W2M_O = 5648
_B2S_O = 8848
_W2S_O = 8880
_B3_O = 12080
_W3_O = 12192
_B4_O = 17568
_W4_O = 17600
_B5_O = 20800
_W5_O = 20912
_B6M_O = 24048
_W6M_O = 24080
_MEM_N = 27280
_BLOB_N = 48 + _MEM_N


def _body(blob_h, out_h, mem, vin, h, r, outv, sem, sem2):
    cid = lax.axis_index("c")
    sid = lax.axis_index("s")

    @pl.when(jnp.logical_and(cid == 0, sid == 0))
    def _():
        iota = lax.iota(jnp.int32, 16)
        tail12 = iota < 12

        ca = pltpu.async_copy(blob_h.at[pl.ds(0, 48)], vin, sem)
        cb = pltpu.async_copy(blob_h.at[pl.ds(48, _B3_O)],
                              mem.at[pl.ds(0, _B3_O)], sem)
        cc = pltpu.async_copy(blob_h.at[pl.ds(48 + _B3_O, _MEM_N - _B3_O)],
                              mem.at[pl.ds(_B3_O, _MEM_N - _B3_O)], sem2)
        ca.wait()
        cb.wait()

        def mm(weights, src, src_n):
            accs = []
            for _w, boff, out_d, _ld in weights:
                nb = (out_d + 15) // 16
                accs += [mem[pl.ds(boff + o * 16, 16)] for o in range(nb)]

            def step(accs, chunk, jl, j):
                bv = jnp.take(chunk, jnp.full((16,), jl, jnp.int32))
                out, k = [], 0
                for woff, _b, out_d, ld in weights:
                    nb = (out_d + 15) // 16
                    row = woff + j * ld
                    for o in range(nb):
                        out.append(accs[k] + bv * mem[pl.ds(row + o * 16, 16)])
                        k += 1
                return tuple(out)

            nchunks, tail = divmod(src_n, 16)

            def cbody(c, accs):
                base = c * 16
                chunk = src[pl.ds(base, 16)]
                for jl in range(16):
                    accs = step(accs, chunk, jl, base + jl)
                return accs

            accs = lax.fori_loop(0, nchunks, cbody, tuple(accs))
            if tail:
                base = nchunks * 16
                chunk = src[pl.ds(base, 16)]
                for jl in range(tail):
                    accs = step(accs, chunk, jl, base + jl)
            res, k = [], 0
            for _w, _b, out_d, _ld in weights:
                nb = (out_d + 15) // 16
                res.append(accs[k:k + nb])
                k += nb
            return res

        def sigm(v):
            return 1.0 / (1.0 + jnp.exp(-v))

        def quant(zv):
            bd = jnp.abs(zv - _CBV[0])
            bv = jnp.full((16,), _CBV[0], dtype=jnp.float32)
            for c in _CBV[1:]:
                d = jnp.abs(zv - jnp.float32(c))
                t = d < bd
                bd = jnp.where(t, d, bd)
                bv = jnp.where(t, jnp.float32(c), bv)
            return bv

        def encode_body(i, carry):
            (hb,) = mm([(_W1_O, _B1_O, 100, 112)], vin, 42)
            for o in range(7):
                h[pl.ds(o * 16, 16)] = jnp.maximum(hb[o], 0.0)
            mres = mm([(_W2M_O, _B2M_O, 28, 32), (_W2S_O, _B2S_O, 28, 32)],
                      h, 100)
            mu0, mu1 = mres[0]
            ls0, ls1 = sigm(mres[1][0]), sigm(mres[1][1])
            mi = jnp.full((16,), i, jnp.int32) < 3
            mbase = 28 + i * 28
            plsc.store_scatter(outv, [mbase + iota], mu0, mask=mi)
            plsc.store_scatter(outv, [mbase + 16 + iota], mu1,
                               mask=jnp.logical_and(mi, tail12))
            lbase = 252 + i * 28
            plsc.store_scatter(outv, [lbase + iota], ls0, mask=mi)
            plsc.store_scatter(outv, [lbase + 16 + iota], ls1,
                               mask=jnp.logical_and(mi, tail12))
            e0 = mem[pl.ds(_EPS_O + i * 32, 16)]
            e1 = mem[pl.ds(_EPS_O + i * 32 + 16, 16)]
            z0, z1 = mu0 + e0 * ls0, mu1 + e1 * ls1
            vin[pl.ds(0, 16)] = quant(z0)
            plsc.store_scatter(vin, [16 + iota], quant(z1), mask=tail12)
            return carry

        lax.fori_loop(0, 4, encode_body, 0)
        cc.wait()

        zero = jnp.zeros((16,), dtype=jnp.float32)
        outv[pl.ds(28 + 84, 16)] = zero
        plsc.store_scatter(outv, [28 + 100 + iota], zero, mask=tail12)
        outv[pl.ds(252 + 84, 16)] = zero
        plsc.store_scatter(outv, [252 + 100 + iota], zero, mask=tail12)

        def dec_body(t, carry):
            k = 3 - t
            (hb,) = mm([(_W3_O, _B3_O, 100, 112)], vin, 42)
            for o in range(7):
                h[pl.ds(o * 16, 16)] = jnp.maximum(hb[o], 0.0)
            ((r0, r1),) = mm([(_W4_O, _B4_O, 28, 32)], h, 100)
            r0, r1 = sigm(r0), sigm(r1)
            r[pl.ds(0, 16)] = r0
            r[pl.ds(16, 16)] = r1
            outv[pl.ds(0, 16)] = r0
            plsc.store_scatter(outv, [16 + iota], r1, mask=tail12)
            vin[pl.ds(0, 16)] = quant(r0)
            plsc.store_scatter(vin, [16 + iota], quant(r1), mask=tail12)
            (hb5,) = mm([(_W5_O, _B5_O, 100, 112)], r, 28)
            for o in range(7):
                h[pl.ds(o * 16, 16)] = hb5[o]
            ((m0, m1),) = mm([(_W6M_O, _B6M_O, 28, 32)], h, 100)
            base = 140 + k * 28
            plsc.store_scatter(outv, [base + iota], m0)
            plsc.store_scatter(outv, [base + 16 + iota], m1, mask=tail12)
            return carry

        lax.fori_loop(0, 4, dec_body, 0)

        pltpu.async_copy(outv, out_h, sem).wait()


_MESH = plsc.VectorSubcoreMesh(core_axis_name="c", subcore_axis_name="s",
                               num_cores=1, num_subcores=1)

_call = pl.kernel(
    _body,
    out_type=[jax.ShapeDtypeStruct((368,), jnp.float32)],
    mesh=_MESH,
    compiler_params=pltpu.CompilerParams(use_tc_tiling_on_sc=False,
                                         needs_layout_passes=False,
                                         disable_bounds_checks=True),
    scratch_types=[
        pltpu.VMEM((_MEM_N,), jnp.float32),
        pltpu.VMEM((48,), jnp.float32),
        pltpu.VMEM((112,), jnp.float32),
        pltpu.VMEM((32,), jnp.float32),
        pltpu.VMEM((368,), jnp.float32),
        pltpu.SemaphoreType.DMA,
        pltpu.SemaphoreType.DMA,
    ],
)


def _tp(W, nrows, ncols):
    out_d, in_d = W.shape
    return jnp.pad(W.T, ((0, nrows - in_d), (0, ncols - out_d))).reshape(-1)


def _tp_cat(W):
    Wt = W.T
    z4 = jnp.zeros((4, 100), jnp.float32)
    z6 = jnp.zeros((6, 100), jnp.float32)
    Wv = jnp.concatenate([Wt[:28], z4, Wt[28:], z6], axis=0)
    return jnp.pad(Wv, ((0, 0), (0, 12))).reshape(-1)


def _padv(v, n):
    return jnp.pad(v, (0, n - v.shape[0]))


def kernel(x, y, params):
    p = params
    blob = jnp.concatenate([
        x, jnp.zeros((4,), jnp.float32), y, jnp.zeros((6,), jnp.float32),
        jnp.asarray(np.pad(_EPS, ((0, 0), (0, 4))).reshape(-1)),
        _padv(p['b1'], 112), _tp_cat(p['W1']),
        _padv(p['b2m'], 32), _tp(p['W2m'], 100, 32),
        _padv(p['b2s'], 32), _tp(p['W2s'], 100, 32),
        _padv(p['b3'], 112), _tp_cat(p['W3']),
        _padv(p['b4'], 32), _tp(p['W4'], 100, 32),
        _padv(p['b5'], 112), _tp(p['W5'], 28, 112),
        _padv(p['b6m'], 32), _tp(p['W6m'], 100, 32),
    ])
    (o,) = _call(blob)
    return (o[0:28], o[28:140].reshape(4, 28), o[140:252].reshape(4, 28),
            o[252:364].reshape(4, 28))
